# baseline (speedup 1.0000x reference)
import functools

import jax
import jax.numpy as jnp
from jax import lax
from jax.experimental import pallas as pl
from jax.experimental.pallas import tpu as pltpu
from jax.experimental.pallas import tpu_sc as plsc

N_NODES = 10000
N_EDGES = 32000
NC = 2
NS = 16
NW = NC * NS
NPADF = 10240
RPT = NPADF // NS
CH = 128
QUOTA = 1024

ROW_BLK = 1000
GRID_M = N_NODES // ROW_BLK
SW = 128


NCH = QUOTA // CH


def _seg_sum_kernel():
    mesh = plsc.VectorSubcoreMesh(core_axis_name="c", subcore_axis_name="s")

    @functools.partial(
        pl.kernel, mesh=mesh,
        out_type=jax.ShapeDtypeStruct((NC, NPADF, SW), jnp.float32),
        scratch_types=[
            pltpu.VMEM((NCH, CH), jnp.int32),
            pltpu.VMEM((NCH, CH), jnp.int32),
            pltpu.VMEM((CH, SW), jnp.float32),
            pltpu.VMEM((CH, SW), jnp.float32),
            pltpu.VMEM_SHARED((NPADF, SW), jnp.float32),
            pltpu.SemaphoreType.DMA,
            pltpu.SemaphoreType.DMA,
        ],
    )
    def seg_sum(p_hbm, src_hbm, dst_hbm, zw_hbm,
                out_hbm, sidx, didx, rv0, rv1, acc_s, sem0, sem1):
        c = lax.axis_index("c")
        s = lax.axis_index("s")
        wid = s * NC + c
        rvs = (rv0, rv1)
        sems = (sem0, sem1)

        pltpu.sync_copy(zw_hbm, rv0)
        for j in range(RPT // CH):
            pltpu.sync_copy(rv0, acc_s.at[pl.ds(RPT * s + CH * j, CH)])
        pltpu.sync_copy(src_hbm.at[wid], sidx)
        pltpu.sync_copy(dst_hbm.at[wid], didx)
        plsc.subcore_barrier()

        def chunk_body(k, _):
            pltpu.async_copy(p_hbm.at[sidx.at[k]], rv0, sem0).wait()
            pltpu.sync_copy(rv0, acc_s.at[didx.at[k]], add=True)
            return 0
        lax.fori_loop(0, NCH, chunk_body, 0)
        plsc.subcore_barrier()

        for j in range(RPT // CH):
            r0 = RPT * s + CH * j
            pltpu.sync_copy(acc_s.at[pl.ds(r0, CH)], rv0)
            pltpu.sync_copy(rv0, out_hbm.at[c, pl.ds(r0, CH)])
        del rv1, rvs, sems, sem1

    return seg_sum


def _seg_cnt_kernel():
    mesh = plsc.VectorSubcoreMesh(core_axis_name="c", subcore_axis_name="s")

    @functools.partial(
        pl.kernel, mesh=mesh,
        out_type=jax.ShapeDtypeStruct((NC, NPADF, SW), jnp.float32),
        scratch_types=[
            pltpu.VMEM((NCH, CH), jnp.int32),
            pltpu.VMEM((CH, SW), jnp.float32),
            pltpu.VMEM_SHARED((NPADF, SW), jnp.float32),
            pltpu.SemaphoreType.DMA,
        ],
    )
    def seg_cnt(dst_hbm, zw_hbm, ow_hbm, out_hbm, didx, ones_v, cnt_s, sem0):
        c = lax.axis_index("c")
        s = lax.axis_index("s")
        wid = s * NC + c

        pltpu.sync_copy(zw_hbm, ones_v)
        for j in range(RPT // CH):
            pltpu.sync_copy(ones_v, cnt_s.at[pl.ds(RPT * s + CH * j, CH)])
        pltpu.sync_copy(dst_hbm.at[wid], didx)
        pltpu.sync_copy(ow_hbm, ones_v)
        plsc.subcore_barrier()

        def chunk_body(k, _):
            pltpu.sync_copy(ones_v, cnt_s.at[didx.at[k]], add=True)
            return 0
        lax.fori_loop(0, NCH, chunk_body, 0)
        plsc.subcore_barrier()
        del sem0

        for j in range(RPT // CH):
            r0 = RPT * s + CH * j
            pltpu.sync_copy(cnt_s.at[pl.ds(r0, CH)], ones_v)
            pltpu.sync_copy(ones_v, out_hbm.at[c, pl.ds(r0, CH)])

    return seg_cnt


_seg_sum = _seg_sum_kernel()
_seg_cnt = _seg_cnt_kernel()

def _psum_spec():
    return pl.BlockSpec((NC, ROW_BLK, SW), lambda i: (0, i, 0))


def _cnt_spec():
    return pl.BlockSpec((NC, ROW_BLK, SW), lambda i: (0, i, 0))


def _full(shape):
    return pl.BlockSpec(shape, lambda i: tuple(0 for _ in shape))


def _rows(W):
    return pl.BlockSpec((ROW_BLK, W), lambda i: (i, 0))


def _inv_deg(cnt_ref):
    csum = cnt_ref[0] + cnt_ref[1]
    return 1.0 / jnp.maximum(csum[:, 0:1], 1.0)


def _k1_body(x_ref, ws_ref, b_ref, wi_ref, o_ref):
    t = jnp.dot(x_ref[...], ws_ref[...], preferred_element_type=jnp.float32)
    t = jnp.maximum(t + b_ref[...], 0.0)
    o_ref[...] = jnp.dot(t, wi_ref[...], preferred_element_type=jnp.float32)


def _k2_body(x_ref, wn_ref, bn_ref, sv_ref, cnt_ref, ws2_ref, b2_ref,
             x1_ref, f2_ref):
    avg = (sv_ref[0] + sv_ref[1]) * _inv_deg(cnt_ref)
    h = jnp.dot(x_ref[...], wn_ref[...], preferred_element_type=jnp.float32)
    x1 = jnp.maximum(h + avg + bn_ref[...], 0.0)
    x1_ref[...] = x1
    h2 = jnp.dot(x1, ws2_ref[...], preferred_element_type=jnp.float32)
    f2_ref[...] = jnp.maximum(h2 + b2_ref[...], 0.0)


def _k3_body(x1_ref, wn2_ref, sv_ref, cnt_ref, wi2_ref, bn2_ref, ws3_ref,
             b3_ref, x2_ref, f3a_ref, f3b_ref):
    avg = (sv_ref[0] + sv_ref[1]) * _inv_deg(cnt_ref)
    h = jnp.dot(x1_ref[...], wn2_ref[...], preferred_element_type=jnp.float32)
    h = h + jnp.dot(avg, wi2_ref[...], preferred_element_type=jnp.float32)
    x2 = jnp.maximum(h + bn2_ref[...], 0.0)
    x2_ref[...] = x2
    h3 = jnp.dot(x2, ws3_ref[...], preferred_element_type=jnp.float32)
    f3 = jnp.maximum(h3 + b3_ref[...], 0.0)
    f3a_ref[...] = f3[:, :128]
    f3b_ref[...] = f3[:, 128:]


def _k4_body(x2_ref, wn3_ref, sva_ref, svb_ref, cnt_ref, wia_ref, wib_ref,
             bn3_ref, wg_ref, bg_ref, o_ref, acc_ref):
    i = pl.program_id(0)
    inv = _inv_deg(cnt_ref)
    avga = (sva_ref[0] + sva_ref[1]) * inv
    avgb = (svb_ref[0] + svb_ref[1]) * inv
    h = jnp.dot(x2_ref[...], wn3_ref[...], preferred_element_type=jnp.float32)
    h = h + jnp.dot(avga, wia_ref[...], preferred_element_type=jnp.float32)
    h = h + jnp.dot(avgb, wib_ref[...], preferred_element_type=jnp.float32)
    x3 = jnp.maximum(h + bn3_ref[...], 0.0)

    @pl.when(i == 0)
    def _():
        acc_ref[...] = jnp.zeros_like(acc_ref)

    acc_ref[...] += jnp.sum(x3, axis=0, keepdims=True)

    @pl.when(i == GRID_M - 1)
    def _():
        m = acc_ref[...] / float(N_NODES)
        o_ref[...] = (jnp.dot(m, wg_ref[...], preferred_element_type=jnp.float32)
                      + bg_ref[...])


def kernel(x, edge_index, Ws1, b1, Wn1, Wi1, bn1, Ws2, b2, Wn2, Wi2, bn2,
           Ws3, b3, Wn3, Wi3, bn3, Wg, bg):
    npad = NW * QUOTA - N_EDGES
    src = jnp.concatenate(
        [edge_index[0], jnp.zeros((npad,), jnp.int32)]).reshape(NW, NCH, CH)
    dst = jnp.concatenate(
        [edge_index[1],
         jnp.full((npad,), N_NODES, jnp.int32)]).reshape(NW, NCH, CH)
    _ZW = jnp.zeros((CH, SW), jnp.float32)
    _OW = jnp.ones((CH, SW), jnp.float32)

    P1 = pl.pallas_call(
        _k1_body,
        grid=(GRID_M,),
        in_specs=[_rows(2048), _full((2048, 2048)), _full((1, 2048)),
                  _full((2048, 128))],
        out_specs=_rows(128),
        out_shape=jax.ShapeDtypeStruct((N_NODES, 128), jnp.float32),
    )(x, Ws1, b1.reshape(1, 2048), Wi1)

    CNT = _seg_cnt(dst, _ZW, _OW)
    S1 = _seg_sum(P1, src, dst, _ZW)

    x1, F2 = pl.pallas_call(
        _k2_body,
        grid=(GRID_M,),
        in_specs=[_rows(2048), _full((2048, 128)), _full((1, 128)),
                  _psum_spec(), _cnt_spec(), _full((128, 128)),
                  _full((1, 128))],
        out_specs=[_rows(128), _rows(128)],
        out_shape=[jax.ShapeDtypeStruct((N_NODES, 128), jnp.float32),
                   jax.ShapeDtypeStruct((N_NODES, 128), jnp.float32)],
    )(x, Wn1, bn1.reshape(1, 128), S1, CNT, Ws2, b2.reshape(1, 128))

    S2 = _seg_sum(F2, src, dst, _ZW)

    x2, F3a, F3b = pl.pallas_call(
        _k3_body,
        grid=(GRID_M,),
        in_specs=[_rows(128), _full((128, 256)), _psum_spec(), _cnt_spec(),
                  _full((128, 256)), _full((1, 256)), _full((256, 256)),
                  _full((1, 256))],
        out_specs=[_rows(256), _rows(128), _rows(128)],
        out_shape=[jax.ShapeDtypeStruct((N_NODES, 256), jnp.float32),
                   jax.ShapeDtypeStruct((N_NODES, 128), jnp.float32),
                   jax.ShapeDtypeStruct((N_NODES, 128), jnp.float32)],
    )(x1, Wn2, S2, CNT, Wi2, bn2.reshape(1, 256), Ws3, b3.reshape(1, 256))

    S3a = _seg_sum(F3a, src, dst, _ZW)
    S3b = _seg_sum(F3b, src, dst, _ZW)

    Wgp = jnp.pad(Wg, ((0, 0), (0, 126)))
    bgp = jnp.pad(bg, (0, 126)).reshape(1, 128)
    g = pl.pallas_call(
        _k4_body,
        grid=(GRID_M,),
        in_specs=[_rows(256), _full((256, 512)), _psum_spec(), _psum_spec(),
                  _cnt_spec(), _full((128, 512)), _full((128, 512)),
                  _full((1, 512)), _full((512, 128)), _full((1, 128))],
        out_specs=_full((1, 128)),
        out_shape=jax.ShapeDtypeStruct((1, 128), jnp.float32),
        scratch_shapes=[pltpu.VMEM((1, 512), jnp.float32)],
    )(x2, Wn3, S3a, S3b, CNT, Wi3[:128], Wi3[128:], bn3.reshape(1, 512),
      Wgp, bgp)

    return g[0, :2]


# speedup vs baseline: 1.0353x; 1.0353x over previous
import functools

import jax
import jax.numpy as jnp
from jax import lax
from jax.experimental import pallas as pl
from jax.experimental.pallas import tpu as pltpu
from jax.experimental.pallas import tpu_sc as plsc

N_NODES = 10000
N_EDGES = 32000
NC = 2
NS = 16
NW = NC * NS
NPADF = 10240
RPT = NPADF // NS
CH = 128
QUOTA = 1024

ROW_BLK = 1000
GRID_M = N_NODES // ROW_BLK
SW = 128


NCH = QUOTA // CH


def _seg_sum_kernel():
    mesh = plsc.VectorSubcoreMesh(core_axis_name="c", subcore_axis_name="s")

    @functools.partial(
        pl.kernel, mesh=mesh,
        out_type=jax.ShapeDtypeStruct((NC, NPADF, SW), jnp.float32),
        scratch_types=[
            pltpu.VMEM((CH,), jnp.int32),
            pltpu.VMEM((CH,), jnp.int32),
            pltpu.VMEM((CH, SW), jnp.float32),
            pltpu.VMEM_SHARED((NPADF, SW), jnp.float32),
            pltpu.SemaphoreType.DMA,
        ],
    )
    def seg_sum(p_hbm, src_hbm, dst_hbm, zw_hbm,
                out_hbm, src_v, idx_v, rows_v, acc_s, sem):
        c = lax.axis_index("c")
        s = lax.axis_index("s")
        wid = s * NC + c

        pltpu.sync_copy(zw_hbm, rows_v)
        for j in range(RPT // CH):
            pltpu.sync_copy(rows_v, acc_s.at[pl.ds(RPT * s + CH * j, CH)])
        plsc.subcore_barrier()

        def chunk_body(k, _):
            base = pl.multiple_of(QUOTA * wid + CH * k, 8)
            pltpu.sync_copy(src_hbm.at[pl.ds(base, CH)], src_v)
            pltpu.sync_copy(dst_hbm.at[pl.ds(base, CH)], idx_v)
            pltpu.async_copy(p_hbm.at[src_v], rows_v, sem).wait()
            pltpu.sync_copy(rows_v, acc_s.at[idx_v], add=True)
            return 0
        lax.fori_loop(0, NCH, chunk_body, 0)
        plsc.subcore_barrier()

        for j in range(RPT // CH):
            r0 = RPT * s + CH * j
            pltpu.sync_copy(acc_s.at[pl.ds(r0, CH)], rows_v)
            pltpu.sync_copy(rows_v, out_hbm.at[c, pl.ds(r0, CH)])

    return seg_sum


def _seg_cnt_kernel():
    mesh = plsc.VectorSubcoreMesh(core_axis_name="c", subcore_axis_name="s")

    @functools.partial(
        pl.kernel, mesh=mesh,
        out_type=jax.ShapeDtypeStruct((NC, NPADF, SW), jnp.float32),
        scratch_types=[
            pltpu.VMEM((CH,), jnp.int32),
            pltpu.VMEM((CH, SW), jnp.float32),
            pltpu.VMEM_SHARED((NPADF, SW), jnp.float32),
        ],
    )
    def seg_cnt(dst_hbm, zw_hbm, ow_hbm, out_hbm, idx_v, ones_v, cnt_s):
        c = lax.axis_index("c")
        s = lax.axis_index("s")
        wid = s * NC + c

        pltpu.sync_copy(zw_hbm, ones_v)
        for j in range(RPT // CH):
            pltpu.sync_copy(ones_v, cnt_s.at[pl.ds(RPT * s + CH * j, CH)])
        pltpu.sync_copy(ow_hbm, ones_v)
        plsc.subcore_barrier()

        def chunk_body(k, _):
            base = pl.multiple_of(QUOTA * wid + CH * k, 8)
            pltpu.sync_copy(dst_hbm.at[pl.ds(base, CH)], idx_v)
            pltpu.sync_copy(ones_v, cnt_s.at[idx_v], add=True)
            return 0
        lax.fori_loop(0, NCH, chunk_body, 0)
        plsc.subcore_barrier()

        for j in range(RPT // CH):
            r0 = RPT * s + CH * j
            pltpu.sync_copy(cnt_s.at[pl.ds(r0, CH)], ones_v)
            pltpu.sync_copy(ones_v, out_hbm.at[c, pl.ds(r0, CH)])

    return seg_cnt


_seg_sum = _seg_sum_kernel()
_seg_cnt = _seg_cnt_kernel()

def _psum_spec():
    return pl.BlockSpec((NC, ROW_BLK, SW), lambda i: (0, i, 0))


def _cnt_spec():
    return pl.BlockSpec((NC, ROW_BLK, SW), lambda i: (0, i, 0))


def _full(shape):
    return pl.BlockSpec(shape, lambda i: tuple(0 for _ in shape))


def _rows(W):
    return pl.BlockSpec((ROW_BLK, W), lambda i: (i, 0))


def _inv_deg(cnt_ref):
    csum = cnt_ref[0] + cnt_ref[1]
    return 1.0 / jnp.maximum(csum[:, 0:1], 1.0)


def _k1_body(x_ref, ws_ref, b_ref, wi_ref, o_ref):
    t = jnp.dot(x_ref[...], ws_ref[...], preferred_element_type=jnp.float32)
    t = jnp.maximum(t + b_ref[...], 0.0)
    o_ref[...] = jnp.dot(t, wi_ref[...], preferred_element_type=jnp.float32)


def _k2_body(x_ref, wn_ref, bn_ref, sv_ref, cnt_ref, ws2_ref, b2_ref,
             x1_ref, f2_ref):
    avg = (sv_ref[0] + sv_ref[1]) * _inv_deg(cnt_ref)
    h = jnp.dot(x_ref[...], wn_ref[...], preferred_element_type=jnp.float32)
    x1 = jnp.maximum(h + avg + bn_ref[...], 0.0)
    x1_ref[...] = x1
    h2 = jnp.dot(x1, ws2_ref[...], preferred_element_type=jnp.float32)
    f2_ref[...] = jnp.maximum(h2 + b2_ref[...], 0.0)


def _k3_body(x1_ref, wn2_ref, sv_ref, cnt_ref, wi2_ref, bn2_ref, ws3_ref,
             b3_ref, x2_ref, f3a_ref, f3b_ref):
    avg = (sv_ref[0] + sv_ref[1]) * _inv_deg(cnt_ref)
    h = jnp.dot(x1_ref[...], wn2_ref[...], preferred_element_type=jnp.float32)
    h = h + jnp.dot(avg, wi2_ref[...], preferred_element_type=jnp.float32)
    x2 = jnp.maximum(h + bn2_ref[...], 0.0)
    x2_ref[...] = x2
    h3 = jnp.dot(x2, ws3_ref[...], preferred_element_type=jnp.float32)
    f3 = jnp.maximum(h3 + b3_ref[...], 0.0)
    f3a_ref[...] = f3[:, :128]
    f3b_ref[...] = f3[:, 128:]


def _k4_body(x2_ref, wn3_ref, sva_ref, svb_ref, cnt_ref, wia_ref, wib_ref,
             bn3_ref, wg_ref, bg_ref, o_ref, acc_ref):
    i = pl.program_id(0)
    inv = _inv_deg(cnt_ref)
    avga = (sva_ref[0] + sva_ref[1]) * inv
    avgb = (svb_ref[0] + svb_ref[1]) * inv
    h = jnp.dot(x2_ref[...], wn3_ref[...], preferred_element_type=jnp.float32)
    h = h + jnp.dot(avga, wia_ref[...], preferred_element_type=jnp.float32)
    h = h + jnp.dot(avgb, wib_ref[...], preferred_element_type=jnp.float32)
    x3 = jnp.maximum(h + bn3_ref[...], 0.0)

    @pl.when(i == 0)
    def _():
        acc_ref[...] = jnp.zeros_like(acc_ref)

    acc_ref[...] += jnp.sum(x3, axis=0, keepdims=True)

    @pl.when(i == GRID_M - 1)
    def _():
        m = acc_ref[...] / float(N_NODES)
        o_ref[...] = (jnp.dot(m, wg_ref[...], preferred_element_type=jnp.float32)
                      + bg_ref[...])


def kernel(x, edge_index, Ws1, b1, Wn1, Wi1, bn1, Ws2, b2, Wn2, Wi2, bn2,
           Ws3, b3, Wn3, Wi3, bn3, Wg, bg):
    npad = NW * QUOTA - N_EDGES
    src = jnp.concatenate(
        [edge_index[0], jnp.zeros((npad,), jnp.int32)])
    dst = jnp.concatenate(
        [edge_index[1], jnp.full((npad,), N_NODES, jnp.int32)])
    _ZW = jnp.zeros((CH, SW), jnp.float32)
    _OW = jnp.ones((CH, SW), jnp.float32)

    P1 = pl.pallas_call(
        _k1_body,
        grid=(GRID_M,),
        in_specs=[_rows(2048), _full((2048, 2048)), _full((1, 2048)),
                  _full((2048, 128))],
        out_specs=_rows(128),
        out_shape=jax.ShapeDtypeStruct((N_NODES, 128), jnp.float32),
    )(x, Ws1, b1.reshape(1, 2048), Wi1)

    CNT = _seg_cnt(dst, _ZW, _OW)
    S1 = _seg_sum(P1, src, dst, _ZW)

    x1, F2 = pl.pallas_call(
        _k2_body,
        grid=(GRID_M,),
        in_specs=[_rows(2048), _full((2048, 128)), _full((1, 128)),
                  _psum_spec(), _cnt_spec(), _full((128, 128)),
                  _full((1, 128))],
        out_specs=[_rows(128), _rows(128)],
        out_shape=[jax.ShapeDtypeStruct((N_NODES, 128), jnp.float32),
                   jax.ShapeDtypeStruct((N_NODES, 128), jnp.float32)],
    )(x, Wn1, bn1.reshape(1, 128), S1, CNT, Ws2, b2.reshape(1, 128))

    S2 = _seg_sum(F2, src, dst, _ZW)

    x2, F3a, F3b = pl.pallas_call(
        _k3_body,
        grid=(GRID_M,),
        in_specs=[_rows(128), _full((128, 256)), _psum_spec(), _cnt_spec(),
                  _full((128, 256)), _full((1, 256)), _full((256, 256)),
                  _full((1, 256))],
        out_specs=[_rows(256), _rows(128), _rows(128)],
        out_shape=[jax.ShapeDtypeStruct((N_NODES, 256), jnp.float32),
                   jax.ShapeDtypeStruct((N_NODES, 128), jnp.float32),
                   jax.ShapeDtypeStruct((N_NODES, 128), jnp.float32)],
    )(x1, Wn2, S2, CNT, Wi2, bn2.reshape(1, 256), Ws3, b3.reshape(1, 256))

    S3a = _seg_sum(F3a, src, dst, _ZW)
    S3b = _seg_sum(F3b, src, dst, _ZW)

    Wgp = jnp.pad(Wg, ((0, 0), (0, 126)))
    bgp = jnp.pad(bg, (0, 126)).reshape(1, 128)
    g = pl.pallas_call(
        _k4_body,
        grid=(GRID_M,),
        in_specs=[_rows(256), _full((256, 512)), _psum_spec(), _psum_spec(),
                  _cnt_spec(), _full((128, 512)), _full((128, 512)),
                  _full((1, 512)), _full((512, 128)), _full((1, 128))],
        out_specs=_full((1, 128)),
        out_shape=jax.ShapeDtypeStruct((1, 128), jnp.float32),
        scratch_shapes=[pltpu.VMEM((1, 512), jnp.float32)],
    )(x2, Wn3, S3a, S3b, CNT, Wi3[:128], Wi3[128:], bn3.reshape(1, 512),
      Wgp, bgp)

    return g[0, :2]


# speedup vs baseline: 1.3523x; 1.3062x over previous
import functools

import jax
import jax.numpy as jnp
from jax import lax
from jax.experimental import pallas as pl
from jax.experimental.pallas import tpu as pltpu
from jax.experimental.pallas import tpu_sc as plsc

N_NODES = 10000
N_EDGES = 32000
NC = 2
NS = 16
NW = NC * NS
NPADF = 10240
RPT = NPADF // NS
CH = 128
QUOTA = 1024

ROW_BLK = 1000
GRID_M = N_NODES // ROW_BLK
SW = 128


NCH = QUOTA // CH


def _seg_sum_kernel():
    mesh = plsc.VectorSubcoreMesh(core_axis_name="c", subcore_axis_name="s")

    @functools.partial(
        pl.kernel, mesh=mesh,
        out_type=jax.ShapeDtypeStruct((NC, NPADF, SW), jnp.float32),
        scratch_types=[
            pltpu.VMEM((CH,), jnp.int32),
            pltpu.VMEM((CH,), jnp.int32),
            pltpu.VMEM((CH, SW), jnp.float32),
            pltpu.VMEM_SHARED((NPADF, SW), jnp.float32),
            pltpu.SemaphoreType.DMA,
        ],
    )
    def seg_sum(p_hbm, src_hbm, dst_hbm, zw_hbm,
                out_hbm, src_v, idx_v, rows_v, acc_s, sem):
        c = lax.axis_index("c")
        s = lax.axis_index("s")
        wid = s * NC + c

        pltpu.sync_copy(zw_hbm, rows_v)
        for j in range(RPT // CH):
            pltpu.sync_copy(rows_v, acc_s.at[pl.ds(RPT * s + CH * j, CH)])
        plsc.subcore_barrier()

        nch = jnp.where(wid == NW - 1, (N_EDGES - (NW - 1) * QUOTA) // CH,
                        QUOTA // CH)

        def chunk_body(k, _):
            @pl.when(k < nch)
            def _():
                base = pl.multiple_of(QUOTA * wid + CH * k, 8)
                pltpu.sync_copy(src_hbm.at[pl.ds(base, CH)], src_v)
                pltpu.sync_copy(dst_hbm.at[pl.ds(base, CH)], idx_v)
                pltpu.async_copy(p_hbm.at[src_v], rows_v, sem).wait()
                pltpu.sync_copy(rows_v, acc_s.at[idx_v], add=True)
            return 0
        lax.fori_loop(0, NCH, chunk_body, 0)
        plsc.subcore_barrier()

        for j in range(RPT // CH):
            r0 = RPT * s + CH * j
            pltpu.sync_copy(acc_s.at[pl.ds(r0, CH)], rows_v)
            pltpu.sync_copy(rows_v, out_hbm.at[c, pl.ds(r0, CH)])

    return seg_sum


def _seg_cnt_kernel():
    mesh = plsc.VectorSubcoreMesh(core_axis_name="c", subcore_axis_name="s")

    @functools.partial(
        pl.kernel, mesh=mesh,
        out_type=jax.ShapeDtypeStruct((NC, NPADF, SW), jnp.float32),
        scratch_types=[
            pltpu.VMEM((CH,), jnp.int32),
            pltpu.VMEM((CH, SW), jnp.float32),
            pltpu.VMEM_SHARED((NPADF, SW), jnp.float32),
        ],
    )
    def seg_cnt(dst_hbm, zw_hbm, ow_hbm, out_hbm, idx_v, ones_v, cnt_s):
        c = lax.axis_index("c")
        s = lax.axis_index("s")
        wid = s * NC + c

        pltpu.sync_copy(zw_hbm, ones_v)
        for j in range(RPT // CH):
            pltpu.sync_copy(ones_v, cnt_s.at[pl.ds(RPT * s + CH * j, CH)])
        pltpu.sync_copy(ow_hbm, ones_v)
        plsc.subcore_barrier()

        nch = jnp.where(wid == NW - 1, (N_EDGES - (NW - 1) * QUOTA) // CH,
                        QUOTA // CH)

        def chunk_body(k, _):
            @pl.when(k < nch)
            def _():
                base = pl.multiple_of(QUOTA * wid + CH * k, 8)
                pltpu.sync_copy(dst_hbm.at[pl.ds(base, CH)], idx_v)
                pltpu.sync_copy(ones_v, cnt_s.at[idx_v], add=True)
            return 0
        lax.fori_loop(0, NCH, chunk_body, 0)
        plsc.subcore_barrier()

        for j in range(RPT // CH):
            r0 = RPT * s + CH * j
            pltpu.sync_copy(cnt_s.at[pl.ds(r0, CH)], ones_v)
            pltpu.sync_copy(ones_v, out_hbm.at[c, pl.ds(r0, CH)])

    return seg_cnt


_seg_sum = _seg_sum_kernel()
_seg_cnt = _seg_cnt_kernel()

def _psum_spec():
    return pl.BlockSpec((NC, ROW_BLK, SW), lambda i: (0, i, 0))


def _cnt_spec():
    return pl.BlockSpec((NC, ROW_BLK, SW), lambda i: (0, i, 0))


def _full(shape):
    return pl.BlockSpec(shape, lambda i: tuple(0 for _ in shape))


def _rows(W):
    return pl.BlockSpec((ROW_BLK, W), lambda i: (i, 0))


def _inv_deg(cnt_ref):
    csum = cnt_ref[0] + cnt_ref[1]
    return 1.0 / jnp.maximum(csum[:, 0:1], 1.0)


def _k1_body(x_ref, ws_ref, b_ref, wi_ref, o_ref):
    t = jnp.dot(x_ref[...], ws_ref[...], preferred_element_type=jnp.float32)
    t = jnp.maximum(t + b_ref[...], 0.0)
    o_ref[...] = jnp.dot(t, wi_ref[...], preferred_element_type=jnp.float32)


def _k2_body(x_ref, wn_ref, bn_ref, sv_ref, cnt_ref, ws2_ref, b2_ref,
             x1_ref, f2_ref):
    avg = (sv_ref[0] + sv_ref[1]) * _inv_deg(cnt_ref)
    h = jnp.dot(x_ref[...], wn_ref[...], preferred_element_type=jnp.float32)
    x1 = jnp.maximum(h + avg + bn_ref[...], 0.0)
    x1_ref[...] = x1
    h2 = jnp.dot(x1, ws2_ref[...], preferred_element_type=jnp.float32)
    f2_ref[...] = jnp.maximum(h2 + b2_ref[...], 0.0)


def _k3_body(x1_ref, wn2_ref, sv_ref, cnt_ref, wi2_ref, bn2_ref, ws3_ref,
             b3_ref, x2_ref, f3a_ref, f3b_ref):
    avg = (sv_ref[0] + sv_ref[1]) * _inv_deg(cnt_ref)
    h = jnp.dot(x1_ref[...], wn2_ref[...], preferred_element_type=jnp.float32)
    h = h + jnp.dot(avg, wi2_ref[...], preferred_element_type=jnp.float32)
    x2 = jnp.maximum(h + bn2_ref[...], 0.0)
    x2_ref[...] = x2
    h3 = jnp.dot(x2, ws3_ref[...], preferred_element_type=jnp.float32)
    f3 = jnp.maximum(h3 + b3_ref[...], 0.0)
    f3a_ref[...] = f3[:, :128]
    f3b_ref[...] = f3[:, 128:]


def _k4_body(x2_ref, wn3_ref, sva_ref, svb_ref, cnt_ref, wia_ref, wib_ref,
             bn3_ref, wg_ref, bg_ref, o_ref, acc_ref):
    i = pl.program_id(0)
    inv = _inv_deg(cnt_ref)
    avga = (sva_ref[0] + sva_ref[1]) * inv
    avgb = (svb_ref[0] + svb_ref[1]) * inv
    h = jnp.dot(x2_ref[...], wn3_ref[...], preferred_element_type=jnp.float32)
    h = h + jnp.dot(avga, wia_ref[...], preferred_element_type=jnp.float32)
    h = h + jnp.dot(avgb, wib_ref[...], preferred_element_type=jnp.float32)
    x3 = jnp.maximum(h + bn3_ref[...], 0.0)

    @pl.when(i == 0)
    def _():
        acc_ref[...] = jnp.zeros_like(acc_ref)

    acc_ref[...] += jnp.sum(x3, axis=0, keepdims=True)

    @pl.when(i == GRID_M - 1)
    def _():
        m = acc_ref[...] / float(N_NODES)
        o_ref[...] = (jnp.dot(m, wg_ref[...], preferred_element_type=jnp.float32)
                      + bg_ref[...])


def kernel(x, edge_index, Ws1, b1, Wn1, Wi1, bn1, Ws2, b2, Wn2, Wi2, bn2,
           Ws3, b3, Wn3, Wi3, bn3, Wg, bg):
    src = edge_index[0]
    dst = edge_index[1]
    _ZW = jnp.zeros((CH, SW), jnp.float32)
    _OW = jnp.ones((CH, SW), jnp.float32)

    P1 = pl.pallas_call(
        _k1_body,
        grid=(GRID_M,),
        in_specs=[_rows(2048), _full((2048, 2048)), _full((1, 2048)),
                  _full((2048, 128))],
        out_specs=_rows(128),
        out_shape=jax.ShapeDtypeStruct((N_NODES, 128), jnp.float32),
    )(x, Ws1, b1.reshape(1, 2048), Wi1)

    CNT = _seg_cnt(dst, _ZW, _OW)
    S1 = _seg_sum(P1, src, dst, _ZW)

    x1, F2 = pl.pallas_call(
        _k2_body,
        grid=(GRID_M,),
        in_specs=[_rows(2048), _full((2048, 128)), _full((1, 128)),
                  _psum_spec(), _cnt_spec(), _full((128, 128)),
                  _full((1, 128))],
        out_specs=[_rows(128), _rows(128)],
        out_shape=[jax.ShapeDtypeStruct((N_NODES, 128), jnp.float32),
                   jax.ShapeDtypeStruct((N_NODES, 128), jnp.float32)],
    )(x, Wn1, bn1.reshape(1, 128), S1, CNT, Ws2, b2.reshape(1, 128))

    S2 = _seg_sum(F2, src, dst, _ZW)

    x2, F3a, F3b = pl.pallas_call(
        _k3_body,
        grid=(GRID_M,),
        in_specs=[_rows(128), _full((128, 256)), _psum_spec(), _cnt_spec(),
                  _full((128, 256)), _full((1, 256)), _full((256, 256)),
                  _full((1, 256))],
        out_specs=[_rows(256), _rows(128), _rows(128)],
        out_shape=[jax.ShapeDtypeStruct((N_NODES, 256), jnp.float32),
                   jax.ShapeDtypeStruct((N_NODES, 128), jnp.float32),
                   jax.ShapeDtypeStruct((N_NODES, 128), jnp.float32)],
    )(x1, Wn2, S2, CNT, Wi2, bn2.reshape(1, 256), Ws3, b3.reshape(1, 256))

    S3a = _seg_sum(F3a, src, dst, _ZW)
    S3b = _seg_sum(F3b, src, dst, _ZW)

    Wgp = jnp.pad(Wg, ((0, 0), (0, 126)))
    bgp = jnp.pad(bg, (0, 126)).reshape(1, 128)
    g = pl.pallas_call(
        _k4_body,
        grid=(GRID_M,),
        in_specs=[_rows(256), _full((256, 512)), _psum_spec(), _psum_spec(),
                  _cnt_spec(), _full((128, 512)), _full((128, 512)),
                  _full((1, 512)), _full((512, 128)), _full((1, 128))],
        out_specs=_full((1, 128)),
        out_shape=jax.ShapeDtypeStruct((1, 128), jnp.float32),
        scratch_shapes=[pltpu.VMEM((1, 512), jnp.float32)],
    )(x2, Wn3, S3a, S3b, CNT, Wi3[:128], Wi3[128:], bn3.reshape(1, 512),
      Wgp, bgp)

    return g[0, :2]


# speedup vs baseline: 1.3651x; 1.0095x over previous
import functools

import jax
import jax.numpy as jnp
from jax import lax
from jax.experimental import pallas as pl
from jax.experimental.pallas import tpu as pltpu
from jax.experimental.pallas import tpu_sc as plsc

N_NODES = 10000
N_EDGES = 32000
NC = 2
NS = 16
NW = NC * NS
NPADF = 10240
RPT = NPADF // NS
CH = 128
QUOTA = 1024

ROW_BLK = 1000
GRID_M = N_NODES // ROW_BLK
SW = 128


NCH = QUOTA // CH


def _seg_sum_kernel():
    mesh = plsc.VectorSubcoreMesh(core_axis_name="c", subcore_axis_name="s")

    @functools.partial(
        pl.kernel, mesh=mesh,
        out_type=jax.ShapeDtypeStruct((NC, NPADF, SW), jnp.float32),
        scratch_types=[
            pltpu.VMEM((CH,), jnp.int32),
            pltpu.VMEM((CH,), jnp.int32),
            pltpu.VMEM((CH, SW), jnp.float32),
            pltpu.VMEM_SHARED((NPADF, SW), jnp.float32),
            pltpu.SemaphoreType.DMA,
        ],
    )
    def seg_sum(p_hbm, src_hbm, dst_hbm, zw_hbm,
                out_hbm, src_v, idx_v, rows_v, acc_s, sem):
        c = lax.axis_index("c")
        s = lax.axis_index("s")
        wid = s * NC + c

        pltpu.sync_copy(zw_hbm, rows_v)
        for j in range(RPT // CH):
            pltpu.sync_copy(rows_v, acc_s.at[pl.ds(RPT * s + CH * j, CH)])
        plsc.subcore_barrier()

        nch = jnp.where(wid == NW - 1, (N_EDGES - (NW - 1) * QUOTA) // CH,
                        QUOTA // CH)

        def chunk_body(k, _):
            @pl.when(k < nch)
            def _():
                base = pl.multiple_of(QUOTA * wid + CH * k, 8)
                pltpu.sync_copy(src_hbm.at[pl.ds(base, CH)], src_v)
                pltpu.sync_copy(dst_hbm.at[pl.ds(base, CH)], idx_v)
                pltpu.async_copy(p_hbm.at[src_v], rows_v, sem).wait()
                pltpu.sync_copy(rows_v, acc_s.at[idx_v], add=True)
            return 0
        lax.fori_loop(0, NCH, chunk_body, 0)
        plsc.subcore_barrier()

        for j in range(RPT // CH):
            r0 = RPT * s + CH * j
            pltpu.sync_copy(acc_s.at[pl.ds(r0, CH)], rows_v)
            pltpu.sync_copy(rows_v, out_hbm.at[c, pl.ds(r0, CH)])

    return seg_sum


def _seg_cnt_kernel():
    mesh = plsc.VectorSubcoreMesh(core_axis_name="c", subcore_axis_name="s")

    @functools.partial(
        pl.kernel, mesh=mesh,
        out_type=jax.ShapeDtypeStruct((NC, NPADF, SW), jnp.float32),
        scratch_types=[
            pltpu.VMEM((CH,), jnp.int32),
            pltpu.VMEM((CH, SW), jnp.float32),
            pltpu.VMEM_SHARED((NPADF, SW), jnp.float32),
        ],
    )
    def seg_cnt(dst_hbm, zw_hbm, ow_hbm, out_hbm, idx_v, ones_v, cnt_s):
        c = lax.axis_index("c")
        s = lax.axis_index("s")
        wid = s * NC + c

        pltpu.sync_copy(zw_hbm, ones_v)
        for j in range(RPT // CH):
            pltpu.sync_copy(ones_v, cnt_s.at[pl.ds(RPT * s + CH * j, CH)])
        pltpu.sync_copy(ow_hbm, ones_v)
        plsc.subcore_barrier()

        nch = jnp.where(wid == NW - 1, (N_EDGES - (NW - 1) * QUOTA) // CH,
                        QUOTA // CH)

        def chunk_body(k, _):
            @pl.when(k < nch)
            def _():
                base = pl.multiple_of(QUOTA * wid + CH * k, 8)
                pltpu.sync_copy(dst_hbm.at[pl.ds(base, CH)], idx_v)
                pltpu.sync_copy(ones_v, cnt_s.at[idx_v], add=True)
            return 0
        lax.fori_loop(0, NCH, chunk_body, 0)
        plsc.subcore_barrier()

        for j in range(RPT // CH):
            r0 = RPT * s + CH * j
            pltpu.sync_copy(cnt_s.at[pl.ds(r0, CH)], ones_v)
            pltpu.sync_copy(ones_v, out_hbm.at[c, pl.ds(r0, CH)])

    return seg_cnt


_seg_sum = _seg_sum_kernel()
_seg_cnt = _seg_cnt_kernel()

def _psum_spec():
    return pl.BlockSpec((NC, ROW_BLK, SW), lambda i: (0, i, 0))


def _cnt_spec():
    return pl.BlockSpec((NC, ROW_BLK, SW), lambda i: (0, i, 0))


def _full(shape):
    return pl.BlockSpec(shape, lambda i: tuple(0 for _ in shape))


def _rows(W):
    return pl.BlockSpec((ROW_BLK, W), lambda i: (i, 0))


def _inv_deg(cnt_ref):
    csum = cnt_ref[0] + cnt_ref[1]
    return 1.0 / jnp.maximum(csum[:, 0:1], 1.0)


def _k1_body(x_ref, ws_ref, b_ref, wi_ref, wn_ref, p_ref, h2_ref):
    xb = x_ref[...].astype(jnp.bfloat16)
    t = jnp.dot(xb, ws_ref[...].astype(jnp.bfloat16),
                preferred_element_type=jnp.float32)
    t = jnp.maximum(t + b_ref[...], 0.0).astype(jnp.bfloat16)
    p_ref[...] = jnp.dot(t, wi_ref[...].astype(jnp.bfloat16),
                         preferred_element_type=jnp.float32)
    h2_ref[...] = jnp.dot(xb, wn_ref[...].astype(jnp.bfloat16),
                          preferred_element_type=jnp.float32)


def _k2_body(h2_ref, sv_ref, cnt_ref, bn_ref, ws2_ref, b2_ref, wn2_ref,
             f2_ref, h3_ref):
    avg = (sv_ref[0] + sv_ref[1]) * _inv_deg(cnt_ref)
    x1 = jnp.maximum(h2_ref[...] + avg + bn_ref[...], 0.0)
    f2_ref[...] = jnp.maximum(
        jnp.dot(x1, ws2_ref[...], preferred_element_type=jnp.float32)
        + b2_ref[...], 0.0)
    h3_ref[...] = jnp.dot(x1, wn2_ref[...],
                          preferred_element_type=jnp.float32)


def _k3_body(h3_ref, sv_ref, cnt_ref, wi2_ref, bn2_ref, ws3_ref, b3_ref,
             wn3_ref, f3a_ref, f3b_ref, h4_ref):
    avg = (sv_ref[0] + sv_ref[1]) * _inv_deg(cnt_ref)
    h = h3_ref[...] + jnp.dot(avg, wi2_ref[...],
                              preferred_element_type=jnp.float32)
    x2 = jnp.maximum(h + bn2_ref[...], 0.0)
    f3 = jnp.maximum(
        jnp.dot(x2, ws3_ref[...], preferred_element_type=jnp.float32)
        + b3_ref[...], 0.0)
    f3a_ref[...] = f3[:, :128]
    f3b_ref[...] = f3[:, 128:]
    h4_ref[...] = jnp.dot(x2, wn3_ref[...],
                          preferred_element_type=jnp.float32)


def _k4_body(h4_ref, sva_ref, svb_ref, cnt_ref, wia_ref, wib_ref,
             bn3_ref, wg_ref, bg_ref, o_ref, acc_ref):
    i = pl.program_id(0)
    inv = _inv_deg(cnt_ref)
    avga = (sva_ref[0] + sva_ref[1]) * inv
    avgb = (svb_ref[0] + svb_ref[1]) * inv
    h = h4_ref[...] + jnp.dot(avga, wia_ref[...],
                              preferred_element_type=jnp.float32)
    h = h + jnp.dot(avgb, wib_ref[...], preferred_element_type=jnp.float32)
    x3 = jnp.maximum(h + bn3_ref[...], 0.0)

    @pl.when(i == 0)
    def _():
        acc_ref[...] = jnp.zeros_like(acc_ref)

    acc_ref[...] += jnp.sum(x3, axis=0, keepdims=True)

    @pl.when(i == GRID_M - 1)
    def _():
        m = acc_ref[...] / float(N_NODES)
        o_ref[...] = (jnp.dot(m, wg_ref[...],
                              preferred_element_type=jnp.float32)
                      + bg_ref[...])


def kernel(x, edge_index, Ws1, b1, Wn1, Wi1, bn1, Ws2, b2, Wn2, Wi2, bn2,
           Ws3, b3, Wn3, Wi3, bn3, Wg, bg):
    src = edge_index[0]
    dst = edge_index[1]
    _ZW = jnp.zeros((CH, SW), jnp.float32)
    _OW = jnp.ones((CH, SW), jnp.float32)

    P1, H2 = pl.pallas_call(
        _k1_body,
        grid=(GRID_M,),
        in_specs=[_rows(2048), _full((2048, 2048)), _full((1, 2048)),
                  _full((2048, 128)), _full((2048, 128))],
        out_specs=[_rows(128), _rows(128)],
        out_shape=[jax.ShapeDtypeStruct((N_NODES, 128), jnp.float32),
                   jax.ShapeDtypeStruct((N_NODES, 128), jnp.float32)],
    )(x, Ws1, b1.reshape(1, 2048), Wi1, Wn1)

    CNT = _seg_cnt(dst, _ZW, _OW)
    S1 = _seg_sum(P1, src, dst, _ZW)

    F2, H3 = pl.pallas_call(
        _k2_body,
        grid=(GRID_M,),
        in_specs=[_rows(128), _psum_spec(), _cnt_spec(), _full((1, 128)),
                  _full((128, 128)), _full((1, 128)), _full((128, 256))],
        out_specs=[_rows(128), _rows(256)],
        out_shape=[jax.ShapeDtypeStruct((N_NODES, 128), jnp.float32),
                   jax.ShapeDtypeStruct((N_NODES, 256), jnp.float32)],
    )(H2, S1, CNT, bn1.reshape(1, 128), Ws2, b2.reshape(1, 128), Wn2)

    S2 = _seg_sum(F2, src, dst, _ZW)

    F3a, F3b, H4 = pl.pallas_call(
        _k3_body,
        grid=(GRID_M,),
        in_specs=[_rows(256), _psum_spec(), _cnt_spec(), _full((128, 256)),
                  _full((1, 256)), _full((256, 256)), _full((1, 256)),
                  _full((256, 512))],
        out_specs=[_rows(128), _rows(128), _rows(512)],
        out_shape=[jax.ShapeDtypeStruct((N_NODES, 128), jnp.float32),
                   jax.ShapeDtypeStruct((N_NODES, 128), jnp.float32),
                   jax.ShapeDtypeStruct((N_NODES, 512), jnp.float32)],
    )(H3, S2, CNT, Wi2, bn2.reshape(1, 256), Ws3, b3.reshape(1, 256), Wn3)

    S3a = _seg_sum(F3a, src, dst, _ZW)
    S3b = _seg_sum(F3b, src, dst, _ZW)

    Wgp = jnp.pad(Wg, ((0, 0), (0, 126)))
    bgp = jnp.pad(bg, (0, 126)).reshape(1, 128)
    g = pl.pallas_call(
        _k4_body,
        grid=(GRID_M,),
        in_specs=[_rows(512), _psum_spec(), _psum_spec(), _cnt_spec(),
                  _full((128, 512)), _full((128, 512)), _full((1, 512)),
                  _full((512, 128)), _full((1, 128))],
        out_specs=_full((1, 128)),
        out_shape=jax.ShapeDtypeStruct((1, 128), jnp.float32),
        scratch_shapes=[pltpu.VMEM((1, 512), jnp.float32)],
    )(H4, S3a, S3b, CNT, Wi3[:128], Wi3[128:], bn3.reshape(1, 512),
      Wgp, bgp)

    return g[0, :2]


# speedup vs baseline: 1.3708x; 1.0041x over previous
import functools

import jax
import jax.numpy as jnp
from jax import lax
from jax.experimental import pallas as pl
from jax.experimental.pallas import tpu as pltpu
from jax.experimental.pallas import tpu_sc as plsc

N_NODES = 10000
N_EDGES = 32000
NC = 2
NS = 16
NW = NC * NS
NPADF = 10240
RPT = NPADF // NS
CH = 128
QUOTA = 1024

ROW_BLK = 1000
GRID_M = N_NODES // ROW_BLK
SW = 128


NCH = QUOTA // CH


def _seg_sum_kernel():
    mesh = plsc.VectorSubcoreMesh(core_axis_name="c", subcore_axis_name="s")

    @functools.partial(
        pl.kernel, mesh=mesh,
        out_type=jax.ShapeDtypeStruct((NC, NPADF, SW), jnp.float32),
        scratch_types=[
            pltpu.VMEM((CH,), jnp.int32),
            pltpu.VMEM((CH,), jnp.int32),
            pltpu.VMEM((CH, SW), jnp.float32),
            pltpu.VMEM_SHARED((NPADF, SW), jnp.float32),
            pltpu.SemaphoreType.DMA,
        ],
    )
    def seg_sum(p_hbm, src_hbm, dst_hbm, zw_hbm,
                out_hbm, src_v, idx_v, rows_v, acc_s, sem):
        c = lax.axis_index("c")
        s = lax.axis_index("s")
        wid = s * NC + c

        pltpu.sync_copy(zw_hbm, rows_v)
        for j in range(RPT // CH):
            pltpu.sync_copy(rows_v, acc_s.at[pl.ds(RPT * s + CH * j, CH)])
        plsc.subcore_barrier()

        nch = jnp.where(wid == NW - 1, (N_EDGES - (NW - 1) * QUOTA) // CH,
                        QUOTA // CH)

        def chunk_body(k, _):
            @pl.when(k < nch)
            def _():
                base = pl.multiple_of(QUOTA * wid + CH * k, 8)
                pltpu.sync_copy(src_hbm.at[pl.ds(base, CH)], src_v)
                pltpu.sync_copy(dst_hbm.at[pl.ds(base, CH)], idx_v)
                pltpu.async_copy(p_hbm.at[src_v], rows_v, sem).wait()
                pltpu.sync_copy(rows_v, acc_s.at[idx_v], add=True)
            return 0
        lax.fori_loop(0, NCH, chunk_body, 0)
        plsc.subcore_barrier()

        for j in range(RPT // CH):
            r0 = RPT * s + CH * j
            pltpu.sync_copy(acc_s.at[pl.ds(r0, CH)], rows_v)
            pltpu.sync_copy(rows_v, out_hbm.at[c, pl.ds(r0, CH)])

    return seg_sum


def _seg_cnt_kernel():
    mesh = plsc.VectorSubcoreMesh(core_axis_name="c", subcore_axis_name="s")

    @functools.partial(
        pl.kernel, mesh=mesh,
        out_type=jax.ShapeDtypeStruct((NC, NPADF, SW), jnp.float32),
        scratch_types=[
            pltpu.VMEM((CH,), jnp.int32),
            pltpu.VMEM((CH, SW), jnp.float32),
            pltpu.VMEM_SHARED((NPADF, SW), jnp.float32),
        ],
    )
    def seg_cnt(dst_hbm, zw_hbm, ow_hbm, out_hbm, idx_v, ones_v, cnt_s):
        c = lax.axis_index("c")
        s = lax.axis_index("s")
        wid = s * NC + c

        pltpu.sync_copy(zw_hbm, ones_v)
        for j in range(RPT // CH):
            pltpu.sync_copy(ones_v, cnt_s.at[pl.ds(RPT * s + CH * j, CH)])
        pltpu.sync_copy(ow_hbm, ones_v)
        plsc.subcore_barrier()

        nch = jnp.where(wid == NW - 1, (N_EDGES - (NW - 1) * QUOTA) // CH,
                        QUOTA // CH)

        def chunk_body(k, _):
            @pl.when(k < nch)
            def _():
                base = pl.multiple_of(QUOTA * wid + CH * k, 8)
                pltpu.sync_copy(dst_hbm.at[pl.ds(base, CH)], idx_v)
                pltpu.sync_copy(ones_v, cnt_s.at[idx_v], add=True)
            return 0
        lax.fori_loop(0, NCH, chunk_body, 0)
        plsc.subcore_barrier()

        for j in range(RPT // CH):
            r0 = RPT * s + CH * j
            pltpu.sync_copy(cnt_s.at[pl.ds(r0, CH)], ones_v)
            pltpu.sync_copy(ones_v, out_hbm.at[c, pl.ds(r0, CH)])

    return seg_cnt


_seg_sum = _seg_sum_kernel()
_seg_cnt = _seg_cnt_kernel()

def _psum_spec():
    return pl.BlockSpec((NC, ROW_BLK, SW), lambda i: (0, i, 0))


def _cnt_spec():
    return pl.BlockSpec((NC, ROW_BLK, SW), lambda i: (0, i, 0))


def _full(shape):
    return pl.BlockSpec(shape, lambda i: tuple(0 for _ in shape))


def _rows(W):
    return pl.BlockSpec((ROW_BLK, W), lambda i: (i, 0))


def _inv_deg(cnt_ref):
    csum = cnt_ref[0] + cnt_ref[1]
    return 1.0 / jnp.maximum(csum[:, 0:1], 1.0)


def _k1_body(x_ref, ws_ref, b_ref, wi_ref, wn_ref, p_ref, h2_ref):
    xv = x_ref[...]
    t = jnp.dot(xv, ws_ref[...], preferred_element_type=jnp.float32)
    t = jnp.maximum(t + b_ref[...], 0.0)
    p_ref[...] = jnp.dot(t, wi_ref[...], preferred_element_type=jnp.float32)
    h2_ref[...] = jnp.dot(xv, wn_ref[...], preferred_element_type=jnp.float32)


def _k2_body(h2_ref, sv_ref, cnt_ref, bn_ref, ws2_ref, b2_ref, wn2_ref,
             f2_ref, h3_ref):
    avg = (sv_ref[0] + sv_ref[1]) * _inv_deg(cnt_ref)
    x1 = jnp.maximum(h2_ref[...] + avg + bn_ref[...], 0.0)
    f2_ref[...] = jnp.maximum(
        jnp.dot(x1, ws2_ref[...], preferred_element_type=jnp.float32)
        + b2_ref[...], 0.0)
    h3_ref[...] = jnp.dot(x1, wn2_ref[...],
                          preferred_element_type=jnp.float32)


def _k3_body(h3_ref, sv_ref, cnt_ref, wi2_ref, bn2_ref, ws3_ref, b3_ref,
             wn3_ref, f3a_ref, f3b_ref, h4_ref):
    avg = (sv_ref[0] + sv_ref[1]) * _inv_deg(cnt_ref)
    h = h3_ref[...] + jnp.dot(avg, wi2_ref[...],
                              preferred_element_type=jnp.float32)
    x2 = jnp.maximum(h + bn2_ref[...], 0.0)
    f3 = jnp.maximum(
        jnp.dot(x2, ws3_ref[...], preferred_element_type=jnp.float32)
        + b3_ref[...], 0.0)
    f3a_ref[...] = f3[:, :128]
    f3b_ref[...] = f3[:, 128:]
    h4_ref[...] = jnp.dot(x2, wn3_ref[...],
                          preferred_element_type=jnp.float32)


def _k4_body(h4_ref, sva_ref, svb_ref, cnt_ref, wia_ref, wib_ref,
             bn3_ref, wg_ref, bg_ref, o_ref, acc_ref):
    i = pl.program_id(0)
    inv = _inv_deg(cnt_ref)
    avga = (sva_ref[0] + sva_ref[1]) * inv
    avgb = (svb_ref[0] + svb_ref[1]) * inv
    h = h4_ref[...] + jnp.dot(avga, wia_ref[...],
                              preferred_element_type=jnp.float32)
    h = h + jnp.dot(avgb, wib_ref[...], preferred_element_type=jnp.float32)
    x3 = jnp.maximum(h + bn3_ref[...], 0.0)

    @pl.when(i == 0)
    def _():
        acc_ref[...] = jnp.zeros_like(acc_ref)

    acc_ref[...] += jnp.sum(x3, axis=0, keepdims=True)

    @pl.when(i == GRID_M - 1)
    def _():
        m = acc_ref[...] / float(N_NODES)
        o_ref[...] = (jnp.dot(m, wg_ref[...],
                              preferred_element_type=jnp.float32)
                      + bg_ref[...])


def kernel(x, edge_index, Ws1, b1, Wn1, Wi1, bn1, Ws2, b2, Wn2, Wi2, bn2,
           Ws3, b3, Wn3, Wi3, bn3, Wg, bg):
    src = edge_index[0]
    dst = edge_index[1]
    _ZW = jnp.zeros((CH, SW), jnp.float32)
    _OW = jnp.ones((CH, SW), jnp.float32)

    P1, H2 = pl.pallas_call(
        _k1_body,
        grid=(GRID_M,),
        in_specs=[_rows(2048), _full((2048, 2048)), _full((1, 2048)),
                  _full((2048, 128)), _full((2048, 128))],
        out_specs=[_rows(128), _rows(128)],
        out_shape=[jax.ShapeDtypeStruct((N_NODES, 128), jnp.float32),
                   jax.ShapeDtypeStruct((N_NODES, 128), jnp.float32)],
    )(x, Ws1, b1.reshape(1, 2048), Wi1, Wn1)

    CNT = _seg_cnt(dst, _ZW, _OW)
    S1 = _seg_sum(P1, src, dst, _ZW)

    F2, H3 = pl.pallas_call(
        _k2_body,
        grid=(GRID_M,),
        in_specs=[_rows(128), _psum_spec(), _cnt_spec(), _full((1, 128)),
                  _full((128, 128)), _full((1, 128)), _full((128, 256))],
        out_specs=[_rows(128), _rows(256)],
        out_shape=[jax.ShapeDtypeStruct((N_NODES, 128), jnp.float32),
                   jax.ShapeDtypeStruct((N_NODES, 256), jnp.float32)],
    )(H2, S1, CNT, bn1.reshape(1, 128), Ws2, b2.reshape(1, 128), Wn2)

    S2 = _seg_sum(F2, src, dst, _ZW)

    F3a, F3b, H4 = pl.pallas_call(
        _k3_body,
        grid=(GRID_M,),
        in_specs=[_rows(256), _psum_spec(), _cnt_spec(), _full((128, 256)),
                  _full((1, 256)), _full((256, 256)), _full((1, 256)),
                  _full((256, 512))],
        out_specs=[_rows(128), _rows(128), _rows(512)],
        out_shape=[jax.ShapeDtypeStruct((N_NODES, 128), jnp.float32),
                   jax.ShapeDtypeStruct((N_NODES, 128), jnp.float32),
                   jax.ShapeDtypeStruct((N_NODES, 512), jnp.float32)],
    )(H3, S2, CNT, Wi2, bn2.reshape(1, 256), Ws3, b3.reshape(1, 256), Wn3)

    S3a = _seg_sum(F3a, src, dst, _ZW)
    S3b = _seg_sum(F3b, src, dst, _ZW)

    Wgp = jnp.pad(Wg, ((0, 0), (0, 126)))
    bgp = jnp.pad(bg, (0, 126)).reshape(1, 128)
    g = pl.pallas_call(
        _k4_body,
        grid=(GRID_M,),
        in_specs=[_rows(512), _psum_spec(), _psum_spec(), _cnt_spec(),
                  _full((128, 512)), _full((128, 512)), _full((1, 512)),
                  _full((512, 128)), _full((1, 128))],
        out_specs=_full((1, 128)),
        out_shape=jax.ShapeDtypeStruct((1, 128), jnp.float32),
        scratch_shapes=[pltpu.VMEM((1, 512), jnp.float32)],
    )(H4, S3a, S3b, CNT, Wi3[:128], Wi3[128:], bn3.reshape(1, 512),
      Wgp, bgp)

    return g[0, :2]
